# baseline (speedup 1.0000x reference)
import functools

import jax
import jax.numpy as jnp
from jax import lax
from jax.experimental import pallas as pl
from jax.experimental.pallas import tpu as pltpu
from jax.experimental.pallas import tpu_sc as plsc

N = 10000
E = 320000
D = 128
G = 64

NPAD = 10240
EPAD = 327680
EROWS = EPAD // 128
NSUB = 16
NCORE = 2
NW = NCORE * NSUB
ERW = EROWS // NW
ROWS_SUB = NPAD // NSUB
DUMMY_DST = N + 100
POOL_ROWS = 72
CNT_ROWS = 80
BCOLS = 64
BRW = (NPAD // BCOLS) // NW

_f32 = jnp.float32
_i32 = jnp.int32


def _sc_mesh():
    return plsc.VectorSubcoreMesh(core_axis_name="c", subcore_axis_name="s")


def _deg_body(dst_hbm, deg0_out, deg1_out, idx_v, ones_v, zero_v, deg_sh):
    c = lax.axis_index("c")
    s = lax.axis_index("s")
    wid = c * NSUB + s

    @pl.loop(0, 8)
    def _(i):
        ones_v[pl.ds(i * 16, 16)] = jnp.ones((16,), _f32)

    @pl.loop(0, ROWS_SUB // 16)
    def _(i):
        zero_v[pl.ds(i * 16, 16)] = jnp.zeros((16,), _f32)

    pltpu.sync_copy(zero_v, deg_sh.at[pl.ds(s * ROWS_SUB, ROWS_SUB)])
    plsc.subcore_barrier()

    pltpu.sync_copy(dst_hbm.at[pl.ds(wid * ERW, ERW)], idx_v)

    @pl.loop(0, ERW)
    def _(t):
        pltpu.sync_copy(ones_v, deg_sh.at[idx_v.at[t]], add=True)

    plsc.subcore_barrier()

    @pl.when(c == 0)
    def _():
        pltpu.sync_copy(deg_sh.at[pl.ds(s * ROWS_SUB, ROWS_SUB)],
                        deg0_out.at[pl.ds(s * ROWS_SUB, ROWS_SUB)])

    @pl.when(c == 1)
    def _():
        pltpu.sync_copy(deg_sh.at[pl.ds(s * ROWS_SUB, ROWS_SUB)],
                        deg1_out.at[pl.ds(s * ROWS_SUB, ROWS_SUB)])


def _sc_degree(dst2d):
    return pl.kernel(
        _deg_body,
        out_type=[
            jax.ShapeDtypeStruct((NPAD,), _f32),
            jax.ShapeDtypeStruct((NPAD,), _f32),
        ],
        mesh=_sc_mesh(),
        scratch_types=[
            pltpu.VMEM((ERW, 128), _i32),
            pltpu.VMEM((128,), _f32),
            pltpu.VMEM((ROWS_SUB,), _f32),
            pltpu.VMEM_SHARED((NPAD,), _f32),
        ],
    )(dst2d)


_NBUF = 4
_CH = 64
_NCHROWS = EPAD // _CH
_DH = D // 2
_IBLK = 8

_CHUNKS_A = 272
_CHUNKS_B = 48


def _agg_ring(eidx_hbm, xs_hbm, acc_sh, idx_v, rows, gsems, ssems, base,
              nchunks):
    for j in range(nchunks // _IBLK):
        pltpu.sync_copy(
            eidx_hbm.at[pl.ds(2 * (base + j * _IBLK), 2 * _IBLK)], idx_v)
        for i in range(_NBUF):
            pltpu.async_copy(xs_hbm.at[idx_v.at[2 * i]], rows[i],
                             gsems.at[i])

        @pl.loop(0, _IBLK, step=_NBUF)
        def _(t):
            for i in range(_NBUF):
                pltpu.make_async_copy(xs_hbm.at[idx_v.at[2 * (t + i)]],
                                      rows[i], gsems.at[i]).wait()
                pltpu.async_copy(rows[i],
                                 acc_sh.at[idx_v.at[2 * (t + i) + 1]],
                                 ssems.at[i], add=True)
            for i in range(_NBUF):
                pltpu.make_async_copy(rows[i],
                                      acc_sh.at[idx_v.at[2 * (t + i) + 1]],
                                      ssems.at[i]).wait()

                @pl.when(t + _NBUF + i < _IBLK)
                def _():
                    pltpu.async_copy(xs_hbm.at[idx_v.at[2 * (t + _NBUF + i)]],
                                     rows[i], gsems.at[i])


def _agg_body(xs_hbm, eidx_hbm, acc_out, idx_v, rows0, rows1, rows2, rows3,
              gsems, ssems, acc_sh):
    c = lax.axis_index("c")
    s = lax.axis_index("s")
    rows = [rows0, rows1, rows2, rows3]

    @pl.loop(0, _CH)
    def _(r):
        @pl.loop(0, 8)
        def _(k):
            rows0[r, pl.ds(k * 16, 16)] = jnp.zeros((16,), _f32)

    for j in range(ROWS_SUB // _CH):
        pltpu.sync_copy(rows0, acc_sh.at[pl.ds(s * ROWS_SUB + j * _CH, _CH)])
    plsc.subcore_barrier()

    @pl.when(c == 0)
    def _():
        _agg_ring(eidx_hbm, xs_hbm, acc_sh, idx_v, rows, gsems, ssems,
                  s * _CHUNKS_A, _CHUNKS_A)

    @pl.when(c == 1)
    def _():
        _agg_ring(eidx_hbm, xs_hbm, acc_sh, idx_v, rows, gsems, ssems,
                  NSUB * _CHUNKS_A + s * _CHUNKS_B, _CHUNKS_B)

    plsc.subcore_barrier()
    pltpu.sync_copy(acc_sh.at[pl.ds(s * ROWS_SUB, ROWS_SUB)],
                    acc_out.at[c, pl.ds(s * ROWS_SUB, ROWS_SUB)])


def _sc_aggregate(xs, eidx2d):
    return pl.kernel(
        _agg_body,
        out_type=jax.ShapeDtypeStruct((NCORE, NPAD, D), _f32),
        mesh=_sc_mesh(),
        scratch_types=[
            pltpu.VMEM((2 * _IBLK, _CH), _i32),
            pltpu.VMEM((_CH, D), _f32),
            pltpu.VMEM((_CH, D), _f32),
            pltpu.VMEM((_CH, D), _f32),
            pltpu.VMEM((_CH, D), _f32),
            pltpu.SemaphoreType.DMA((_NBUF,)),
            pltpu.SemaphoreType.DMA((_NBUF,)),
            pltpu.VMEM_SHARED((NPAD, D), _f32),
        ],
    )(xs, eidx2d)


def _pool_body(h_hbm, b_hbm, pool_out, cnt0_out, cnt1_out, bidx_v, rows_v,
               ones_v, z_v, pool_sh, cnt_sh):
    c = lax.axis_index("c")
    s = lax.axis_index("s")
    wid = c * NSUB + s

    @pl.loop(0, BCOLS // 16)
    def _(i):
        ones_v[pl.ds(i * 16, 16)] = jnp.ones((16,), _f32)

    @pl.loop(0, CNT_ROWS // 16)
    def _(i):
        z_v[pl.ds(i * 16, 16)] = jnp.zeros((16,), _f32)

    @pl.when(s == 0)
    def _():
        @pl.loop(0, BCOLS)
        def _(r):
            @pl.loop(0, 8)
            def _(k):
                rows_v[r, pl.ds(k * 16, 16)] = jnp.zeros((16,), _f32)

        pltpu.sync_copy(rows_v, pool_sh.at[pl.ds(0, BCOLS)])
        pltpu.sync_copy(rows_v.at[pl.ds(0, POOL_ROWS - BCOLS)],
                        pool_sh.at[pl.ds(BCOLS, POOL_ROWS - BCOLS)])
        pltpu.sync_copy(z_v, cnt_sh)

    plsc.subcore_barrier()

    pltpu.sync_copy(b_hbm.at[wid], bidx_v)

    for k in range(BRW):
        nb = wid * (BRW * BCOLS) + k * BCOLS
        pltpu.sync_copy(h_hbm.at[pl.ds(nb, BCOLS)], rows_v)
        pltpu.sync_copy(rows_v, pool_sh.at[bidx_v.at[k]], add=True)
        pltpu.sync_copy(ones_v, cnt_sh.at[bidx_v.at[k]], add=True)

    plsc.subcore_barrier()

    @pl.when(jnp.logical_and(s == 0, c == 0))
    def _():
        pltpu.sync_copy(pool_sh, pool_out.at[0])
        pltpu.sync_copy(cnt_sh, cnt0_out)

    @pl.when(jnp.logical_and(s == 0, c == 1))
    def _():
        pltpu.sync_copy(pool_sh, pool_out.at[1])
        pltpu.sync_copy(cnt_sh, cnt1_out)


def _sc_pool(h2, batch3d):
    return pl.kernel(
        _pool_body,
        out_type=[
            jax.ShapeDtypeStruct((NCORE, POOL_ROWS, D), _f32),
            jax.ShapeDtypeStruct((CNT_ROWS,), _f32),
            jax.ShapeDtypeStruct((CNT_ROWS,), _f32),
        ],
        mesh=_sc_mesh(),
        scratch_types=[
            pltpu.VMEM((BRW, BCOLS), _i32),
            pltpu.VMEM((BCOLS, D), _f32),
            pltpu.VMEM((BCOLS,), _f32),
            pltpu.VMEM((CNT_ROWS,), _f32),
            pltpu.VMEM_SHARED((POOL_ROWS, D), _f32),
            pltpu.VMEM_SHARED((CNT_ROWS,), _f32),
        ],
    )(h2, batch3d)


_NBLK = 8
_BLK = NPAD // _NBLK


def _dot(a, b):
    return lax.dot_general(a, b, (((1,), (0,)), ((), ())),
                           preferred_element_type=_f32,
                           precision=lax.Precision.HIGHEST)


def _mm_body(x_ref, w_ref, o_ref):
    o_ref[...] = _dot(x_ref[...], w_ref[...])


def _tc_matmul(x, w):
    return pl.pallas_call(
        _mm_body,
        grid=(_NBLK,),
        in_specs=[
            pl.BlockSpec((_BLK, D), lambda i: (i, 0)),
            pl.BlockSpec((D, D), lambda i: (0, 0)),
        ],
        out_specs=pl.BlockSpec((_BLK, D), lambda i: (i, 0)),
        out_shape=jax.ShapeDtypeStruct((NPAD, D), _f32),
    )(x, w)


def _scale_body(xw_ref, deg_ref, xs_ref, dinv_ref):
    deg = deg_ref[0] + deg_ref[1] + 1.0
    dinv = lax.rsqrt(deg)
    dinv_ref[...] = dinv
    xs_ref[...] = xw_ref[...] * dinv


def _tc_scale(xw, deg_pair):
    return pl.pallas_call(
        _scale_body,
        grid=(_NBLK,),
        in_specs=[
            pl.BlockSpec((_BLK, D), lambda i: (i, 0)),
            pl.BlockSpec((NCORE, _BLK, 1), lambda i: (0, i, 0)),
        ],
        out_specs=[
            pl.BlockSpec((_BLK, D), lambda i: (i, 0)),
            pl.BlockSpec((_BLK, 1), lambda i: (i, 0)),
        ],
        out_shape=[
            jax.ShapeDtypeStruct((NPAD, D), _f32),
            jax.ShapeDtypeStruct((NPAD, 1), _f32),
        ],
    )(xw, deg_pair)


def _mid_body(acc_ref, xs_ref, dinv_ref, b_ref, w_ref, xs2_ref):
    dinv = dinv_ref[...]
    h = (acc_ref[0] + acc_ref[1] + xs_ref[...]) * dinv + b_ref[...]
    h = jnp.maximum(h, 0.0)
    xs2_ref[...] = _dot(h, w_ref[...]) * dinv


def _tc_mid(acc, xs, dinv, b, w):
    return pl.pallas_call(
        _mid_body,
        grid=(_NBLK,),
        in_specs=[
            pl.BlockSpec((NCORE, _BLK, D), lambda i: (0, i, 0)),
            pl.BlockSpec((_BLK, D), lambda i: (i, 0)),
            pl.BlockSpec((_BLK, 1), lambda i: (i, 0)),
            pl.BlockSpec((1, D), lambda i: (0, 0)),
            pl.BlockSpec((D, D), lambda i: (0, 0)),
        ],
        out_specs=pl.BlockSpec((_BLK, D), lambda i: (i, 0)),
        out_shape=jax.ShapeDtypeStruct((NPAD, D), _f32),
    )(acc, xs, dinv, b, w)


def _final_node_body(acc_ref, xs_ref, dinv_ref, b_ref, h_ref):
    h = (acc_ref[0] + acc_ref[1] + xs_ref[...]) * dinv_ref[...] + b_ref[...]
    h_ref[...] = jnp.maximum(h, 0.0)


def _tc_final_nodes(acc, xs, dinv, b):
    return pl.pallas_call(
        _final_node_body,
        grid=(_NBLK,),
        in_specs=[
            pl.BlockSpec((NCORE, _BLK, D), lambda i: (0, i, 0)),
            pl.BlockSpec((_BLK, D), lambda i: (i, 0)),
            pl.BlockSpec((_BLK, 1), lambda i: (i, 0)),
            pl.BlockSpec((1, D), lambda i: (0, 0)),
        ],
        out_specs=pl.BlockSpec((_BLK, D), lambda i: (i, 0)),
        out_shape=jax.ShapeDtypeStruct((NPAD, D), _f32),
    )(acc, xs, dinv, b)


def _head_body(pool_ref, cnt_ref, wl_ref, bl_ref, y_ref):
    ps = pool_ref[0, :G, :] + pool_ref[1, :G, :]
    cn = cnt_ref[0, :G, :] + cnt_ref[1, :G, :]
    cn = jnp.maximum(cn, 1.0)
    pm = ps / cn
    y_ref[...] = _dot(pm, wl_ref[...]) + bl_ref[...]


def _tc_head(pool, cnt, wl, bl):
    return pl.pallas_call(
        _head_body,
        out_shape=jax.ShapeDtypeStruct((G, 2), _f32),
    )(pool, cnt, wl, bl)


def kernel(x, edge_index, batch, W1, b1, W2, b2, Wl, bl):
    src = edge_index[0].astype(_i32)
    dst = edge_index[1].astype(_i32)
    src_p = jnp.concatenate([src, jnp.zeros((EPAD - E,), _i32)])
    dst_p = jnp.concatenate([dst, jnp.full((EPAD - E,), DUMMY_DST, _i32)])
    dst2d = dst_p.reshape(EROWS, 128)
    eidx2d = jnp.stack(
        [src_p.reshape(_NCHROWS, _CH), dst_p.reshape(_NCHROWS, _CH)],
        axis=1).reshape(2 * _NCHROWS, _CH)
    batch3d = jnp.concatenate(
        [batch.astype(_i32), jnp.full((NPAD - N,), G, _i32)]
    ).reshape(NW, BRW, BCOLS)
    x_pad = jnp.pad(x, ((0, NPAD - N), (0, 0)))

    deg0, deg1 = _sc_degree(dst2d)
    xw1 = _tc_matmul(x_pad, W1)
    deg3 = jnp.stack([deg0, deg1]).reshape(NCORE, NPAD, 1)
    xs1, dinv = _tc_scale(xw1, deg3)

    acc1 = _sc_aggregate(xs1, eidx2d)
    xs2 = _tc_mid(acc1, xs1, dinv, b1.reshape(1, D), W2)

    acc2 = _sc_aggregate(xs2, eidx2d)
    h2 = _tc_final_nodes(acc2, xs2, dinv, b2.reshape(1, D))

    pool, cnt0, cnt1 = _sc_pool(h2, batch3d)
    cnt = jnp.stack([cnt0, cnt1]).reshape(NCORE, CNT_ROWS, 1)
    y = _tc_head(pool, cnt, Wl, bl.reshape(1, 2))
    return y


# speedup vs baseline: 1.0978x; 1.0978x over previous
import functools

import jax
import jax.numpy as jnp
from jax import lax
from jax.experimental import pallas as pl
from jax.experimental.pallas import tpu as pltpu
from jax.experimental.pallas import tpu_sc as plsc

N = 10000
E = 320000
D = 128
G = 64

NPAD = 10240
EPAD = 327680
EROWS = EPAD // 128
NSUB = 16
NCORE = 2
NW = NCORE * NSUB
ERW = EROWS // NW
ROWS_SUB = NPAD // NSUB
DUMMY_DST = N + 100
POOL_ROWS = 72
CNT_ROWS = 80
BCOLS = 64
BRW = (NPAD // BCOLS) // NW

_f32 = jnp.float32
_i32 = jnp.int32


def _sc_mesh():
    return plsc.VectorSubcoreMesh(core_axis_name="c", subcore_axis_name="s")


def _deg_body(dst_hbm, deg0_out, deg1_out, idx_v, ones_v, zero_v, deg_sh):
    c = lax.axis_index("c")
    s = lax.axis_index("s")
    wid = c * NSUB + s

    @pl.loop(0, 8)
    def _(i):
        ones_v[pl.ds(i * 16, 16)] = jnp.ones((16,), _f32)

    @pl.loop(0, ROWS_SUB // 16)
    def _(i):
        zero_v[pl.ds(i * 16, 16)] = jnp.zeros((16,), _f32)

    pltpu.sync_copy(zero_v, deg_sh.at[pl.ds(s * ROWS_SUB, ROWS_SUB)])
    plsc.subcore_barrier()

    pltpu.sync_copy(dst_hbm.at[pl.ds(wid * ERW, ERW)], idx_v)

    @pl.loop(0, ERW)
    def _(t):
        pltpu.sync_copy(ones_v, deg_sh.at[idx_v.at[t]], add=True)

    plsc.subcore_barrier()

    @pl.when(c == 0)
    def _():
        pltpu.sync_copy(deg_sh.at[pl.ds(s * ROWS_SUB, ROWS_SUB)],
                        deg0_out.at[pl.ds(s * ROWS_SUB, ROWS_SUB)])

    @pl.when(c == 1)
    def _():
        pltpu.sync_copy(deg_sh.at[pl.ds(s * ROWS_SUB, ROWS_SUB)],
                        deg1_out.at[pl.ds(s * ROWS_SUB, ROWS_SUB)])


def _sc_degree(dst2d):
    return pl.kernel(
        _deg_body,
        out_type=[
            jax.ShapeDtypeStruct((NPAD,), _f32),
            jax.ShapeDtypeStruct((NPAD,), _f32),
        ],
        mesh=_sc_mesh(),
        scratch_types=[
            pltpu.VMEM((ERW, 128), _i32),
            pltpu.VMEM((128,), _f32),
            pltpu.VMEM((ROWS_SUB,), _f32),
            pltpu.VMEM_SHARED((NPAD,), _f32),
        ],
    )(dst2d)


_NBUF = 2
_CH = 128
_NCHROWS = EPAD // _CH
_DH = D // 2
_IBLK = 8

_CHUNKS_A = 144
_CHUNKS_B = 16


def _agg_ring(eidx_hbm, xs_hbm, acc_sh, idx_v, rows, gsems, ssems, base,
              nchunks):
    for j in range(nchunks // _IBLK):
        pltpu.sync_copy(
            eidx_hbm.at[pl.ds(2 * (base + j * _IBLK), 2 * _IBLK)], idx_v)
        for i in range(_NBUF):
            pltpu.async_copy(xs_hbm.at[idx_v.at[2 * i]], rows[i],
                             gsems.at[i])

        @pl.loop(0, _IBLK, step=_NBUF)
        def _(t):
            for i in range(_NBUF):
                pltpu.make_async_copy(xs_hbm.at[idx_v.at[2 * (t + i)]],
                                      rows[i], gsems.at[i]).wait()
                pltpu.async_copy(rows[i],
                                 acc_sh.at[idx_v.at[2 * (t + i) + 1]],
                                 ssems.at[i], add=True)
            for i in range(_NBUF):
                pltpu.make_async_copy(rows[i],
                                      acc_sh.at[idx_v.at[2 * (t + i) + 1]],
                                      ssems.at[i]).wait()

                @pl.when(t + _NBUF + i < _IBLK)
                def _():
                    pltpu.async_copy(xs_hbm.at[idx_v.at[2 * (t + _NBUF + i)]],
                                     rows[i], gsems.at[i])


def _agg_body(xs_hbm, eidx_hbm, acc_out, idx_v, rows0, rows1, gsems, ssems,
              acc_sh):
    c = lax.axis_index("c")
    s = lax.axis_index("s")
    rows = [rows0, rows1]

    @pl.loop(0, _CH)
    def _(r):
        @pl.loop(0, 8)
        def _(k):
            rows0[r, pl.ds(k * 16, 16)] = jnp.zeros((16,), _f32)

    for j in range(ROWS_SUB // _CH):
        pltpu.sync_copy(rows0, acc_sh.at[pl.ds(s * ROWS_SUB + j * _CH, _CH)])
    plsc.subcore_barrier()

    @pl.when(c == 0)
    def _():
        _agg_ring(eidx_hbm, xs_hbm, acc_sh, idx_v, rows, gsems, ssems,
                  s * _CHUNKS_A, _CHUNKS_A)

    @pl.when(c == 1)
    def _():
        _agg_ring(eidx_hbm, xs_hbm, acc_sh, idx_v, rows, gsems, ssems,
                  NSUB * _CHUNKS_A + s * _CHUNKS_B, _CHUNKS_B)

    plsc.subcore_barrier()
    pltpu.sync_copy(acc_sh.at[pl.ds(s * ROWS_SUB, ROWS_SUB)],
                    acc_out.at[c, pl.ds(s * ROWS_SUB, ROWS_SUB)])


def _sc_aggregate(xs, eidx2d):
    return pl.kernel(
        _agg_body,
        out_type=jax.ShapeDtypeStruct((NCORE, NPAD, D), _f32),
        mesh=_sc_mesh(),
        scratch_types=[
            pltpu.VMEM((2 * _IBLK, _CH), _i32),
            pltpu.VMEM((_CH, D), _f32),
            pltpu.VMEM((_CH, D), _f32),
            pltpu.SemaphoreType.DMA((_NBUF,)),
            pltpu.SemaphoreType.DMA((_NBUF,)),
            pltpu.VMEM_SHARED((NPAD, D), _f32),
        ],
    )(xs, eidx2d)


def _pool_body(h_hbm, b_hbm, pool_out, cnt0_out, cnt1_out, bidx_v, rows_v,
               ones_v, z_v, pool_sh, cnt_sh):
    c = lax.axis_index("c")
    s = lax.axis_index("s")
    wid = c * NSUB + s

    @pl.loop(0, BCOLS // 16)
    def _(i):
        ones_v[pl.ds(i * 16, 16)] = jnp.ones((16,), _f32)

    @pl.loop(0, CNT_ROWS // 16)
    def _(i):
        z_v[pl.ds(i * 16, 16)] = jnp.zeros((16,), _f32)

    @pl.when(s == 0)
    def _():
        @pl.loop(0, BCOLS)
        def _(r):
            @pl.loop(0, 8)
            def _(k):
                rows_v[r, pl.ds(k * 16, 16)] = jnp.zeros((16,), _f32)

        pltpu.sync_copy(rows_v, pool_sh.at[pl.ds(0, BCOLS)])
        pltpu.sync_copy(rows_v.at[pl.ds(0, POOL_ROWS - BCOLS)],
                        pool_sh.at[pl.ds(BCOLS, POOL_ROWS - BCOLS)])
        pltpu.sync_copy(z_v, cnt_sh)

    plsc.subcore_barrier()

    pltpu.sync_copy(b_hbm.at[wid], bidx_v)

    for k in range(BRW):
        nb = wid * (BRW * BCOLS) + k * BCOLS
        pltpu.sync_copy(h_hbm.at[pl.ds(nb, BCOLS)], rows_v)
        pltpu.sync_copy(rows_v, pool_sh.at[bidx_v.at[k]], add=True)
        pltpu.sync_copy(ones_v, cnt_sh.at[bidx_v.at[k]], add=True)

    plsc.subcore_barrier()

    @pl.when(jnp.logical_and(s == 0, c == 0))
    def _():
        pltpu.sync_copy(pool_sh, pool_out.at[0])
        pltpu.sync_copy(cnt_sh, cnt0_out)

    @pl.when(jnp.logical_and(s == 0, c == 1))
    def _():
        pltpu.sync_copy(pool_sh, pool_out.at[1])
        pltpu.sync_copy(cnt_sh, cnt1_out)


def _sc_pool(h2, batch3d):
    return pl.kernel(
        _pool_body,
        out_type=[
            jax.ShapeDtypeStruct((NCORE, POOL_ROWS, D), _f32),
            jax.ShapeDtypeStruct((CNT_ROWS,), _f32),
            jax.ShapeDtypeStruct((CNT_ROWS,), _f32),
        ],
        mesh=_sc_mesh(),
        scratch_types=[
            pltpu.VMEM((BRW, BCOLS), _i32),
            pltpu.VMEM((BCOLS, D), _f32),
            pltpu.VMEM((BCOLS,), _f32),
            pltpu.VMEM((CNT_ROWS,), _f32),
            pltpu.VMEM_SHARED((POOL_ROWS, D), _f32),
            pltpu.VMEM_SHARED((CNT_ROWS,), _f32),
        ],
    )(h2, batch3d)


_NBLK = 8
_BLK = NPAD // _NBLK


def _dot(a, b):
    return lax.dot_general(a, b, (((1,), (0,)), ((), ())),
                           preferred_element_type=_f32,
                           precision=lax.Precision.HIGHEST)


def _mm_body(x_ref, w_ref, o_ref):
    o_ref[...] = _dot(x_ref[...], w_ref[...])


def _tc_matmul(x, w):
    return pl.pallas_call(
        _mm_body,
        grid=(_NBLK,),
        in_specs=[
            pl.BlockSpec((_BLK, D), lambda i: (i, 0)),
            pl.BlockSpec((D, D), lambda i: (0, 0)),
        ],
        out_specs=pl.BlockSpec((_BLK, D), lambda i: (i, 0)),
        out_shape=jax.ShapeDtypeStruct((NPAD, D), _f32),
    )(x, w)


def _scale_body(xw_ref, deg_ref, xs_ref, dinv_ref):
    deg = deg_ref[0] + deg_ref[1] + 1.0
    dinv = lax.rsqrt(deg)
    dinv_ref[...] = dinv
    xs_ref[...] = xw_ref[...] * dinv


def _tc_scale(xw, deg_pair):
    return pl.pallas_call(
        _scale_body,
        grid=(_NBLK,),
        in_specs=[
            pl.BlockSpec((_BLK, D), lambda i: (i, 0)),
            pl.BlockSpec((NCORE, _BLK, 1), lambda i: (0, i, 0)),
        ],
        out_specs=[
            pl.BlockSpec((_BLK, D), lambda i: (i, 0)),
            pl.BlockSpec((_BLK, 1), lambda i: (i, 0)),
        ],
        out_shape=[
            jax.ShapeDtypeStruct((NPAD, D), _f32),
            jax.ShapeDtypeStruct((NPAD, 1), _f32),
        ],
    )(xw, deg_pair)


def _mid_body(acc_ref, xs_ref, dinv_ref, b_ref, w_ref, xs2_ref):
    dinv = dinv_ref[...]
    h = (acc_ref[0] + acc_ref[1] + xs_ref[...]) * dinv + b_ref[...]
    h = jnp.maximum(h, 0.0)
    xs2_ref[...] = _dot(h, w_ref[...]) * dinv


def _tc_mid(acc, xs, dinv, b, w):
    return pl.pallas_call(
        _mid_body,
        grid=(_NBLK,),
        in_specs=[
            pl.BlockSpec((NCORE, _BLK, D), lambda i: (0, i, 0)),
            pl.BlockSpec((_BLK, D), lambda i: (i, 0)),
            pl.BlockSpec((_BLK, 1), lambda i: (i, 0)),
            pl.BlockSpec((1, D), lambda i: (0, 0)),
            pl.BlockSpec((D, D), lambda i: (0, 0)),
        ],
        out_specs=pl.BlockSpec((_BLK, D), lambda i: (i, 0)),
        out_shape=jax.ShapeDtypeStruct((NPAD, D), _f32),
    )(acc, xs, dinv, b, w)


def _final_node_body(acc_ref, xs_ref, dinv_ref, b_ref, h_ref):
    h = (acc_ref[0] + acc_ref[1] + xs_ref[...]) * dinv_ref[...] + b_ref[...]
    h_ref[...] = jnp.maximum(h, 0.0)


def _tc_final_nodes(acc, xs, dinv, b):
    return pl.pallas_call(
        _final_node_body,
        grid=(_NBLK,),
        in_specs=[
            pl.BlockSpec((NCORE, _BLK, D), lambda i: (0, i, 0)),
            pl.BlockSpec((_BLK, D), lambda i: (i, 0)),
            pl.BlockSpec((_BLK, 1), lambda i: (i, 0)),
            pl.BlockSpec((1, D), lambda i: (0, 0)),
        ],
        out_specs=pl.BlockSpec((_BLK, D), lambda i: (i, 0)),
        out_shape=jax.ShapeDtypeStruct((NPAD, D), _f32),
    )(acc, xs, dinv, b)


def _head_body(pool_ref, cnt_ref, wl_ref, bl_ref, y_ref):
    ps = pool_ref[0, :G, :] + pool_ref[1, :G, :]
    cn = cnt_ref[0, :G, :] + cnt_ref[1, :G, :]
    cn = jnp.maximum(cn, 1.0)
    pm = ps / cn
    y_ref[...] = _dot(pm, wl_ref[...]) + bl_ref[...]


def _tc_head(pool, cnt, wl, bl):
    return pl.pallas_call(
        _head_body,
        out_shape=jax.ShapeDtypeStruct((G, 2), _f32),
    )(pool, cnt, wl, bl)


def kernel(x, edge_index, batch, W1, b1, W2, b2, Wl, bl):
    src = edge_index[0].astype(_i32)
    dst = edge_index[1].astype(_i32)
    src_p = jnp.concatenate([src, jnp.zeros((EPAD - E,), _i32)])
    dst_p = jnp.concatenate([dst, jnp.full((EPAD - E,), DUMMY_DST, _i32)])
    dst2d = dst_p.reshape(EROWS, 128)
    eidx2d = jnp.stack(
        [src_p.reshape(_NCHROWS, _CH), dst_p.reshape(_NCHROWS, _CH)],
        axis=1).reshape(2 * _NCHROWS, _CH)
    batch3d = jnp.concatenate(
        [batch.astype(_i32), jnp.full((NPAD - N,), G, _i32)]
    ).reshape(NW, BRW, BCOLS)
    x_pad = jnp.pad(x, ((0, NPAD - N), (0, 0)))

    deg0, deg1 = _sc_degree(dst2d)
    xw1 = _tc_matmul(x_pad, W1)
    deg3 = jnp.stack([deg0, deg1]).reshape(NCORE, NPAD, 1)
    xs1, dinv = _tc_scale(xw1, deg3)

    acc1 = _sc_aggregate(xs1, eidx2d)
    xs2 = _tc_mid(acc1, xs1, dinv, b1.reshape(1, D), W2)

    acc2 = _sc_aggregate(xs2, eidx2d)
    h2 = _tc_final_nodes(acc2, xs2, dinv, b2.reshape(1, D))

    pool, cnt0, cnt1 = _sc_pool(h2, batch3d)
    cnt = jnp.stack([cnt0, cnt1]).reshape(NCORE, CNT_ROWS, 1)
    y = _tc_head(pool, cnt, Wl, bl.reshape(1, 2))
    return y


# speedup vs baseline: 1.1025x; 1.0043x over previous
import functools

import jax
import jax.numpy as jnp
from jax import lax
from jax.experimental import pallas as pl
from jax.experimental.pallas import tpu as pltpu
from jax.experimental.pallas import tpu_sc as plsc

N = 10000
E = 320000
D = 128
G = 64

NPAD = 10240
EPAD = 327680
EROWS = EPAD // 128
NSUB = 16
NCORE = 2
NW = NCORE * NSUB
ERW = EROWS // NW
ROWS_SUB = NPAD // NSUB
DUMMY_DST = N + 100
POOL_ROWS = 72
CNT_ROWS = 80
BCOLS = 64
BRW = (NPAD // BCOLS) // NW

_f32 = jnp.float32
_i32 = jnp.int32


def _sc_mesh():
    return plsc.VectorSubcoreMesh(core_axis_name="c", subcore_axis_name="s")


def _deg_body(dst_hbm, deg0_out, deg1_out, idx_v, ones_v, zero_v, deg_sh):
    c = lax.axis_index("c")
    s = lax.axis_index("s")
    wid = c * NSUB + s

    @pl.loop(0, 8)
    def _(i):
        ones_v[pl.ds(i * 16, 16)] = jnp.ones((16,), _f32)

    @pl.loop(0, ROWS_SUB // 16)
    def _(i):
        zero_v[pl.ds(i * 16, 16)] = jnp.zeros((16,), _f32)

    pltpu.sync_copy(zero_v, deg_sh.at[pl.ds(s * ROWS_SUB, ROWS_SUB)])
    plsc.subcore_barrier()

    pltpu.sync_copy(dst_hbm.at[pl.ds(wid * ERW, ERW)], idx_v)

    @pl.loop(0, ERW)
    def _(t):
        pltpu.sync_copy(ones_v, deg_sh.at[idx_v.at[t]], add=True)

    plsc.subcore_barrier()

    @pl.when(c == 0)
    def _():
        pltpu.sync_copy(deg_sh.at[pl.ds(s * ROWS_SUB, ROWS_SUB)],
                        deg0_out.at[pl.ds(s * ROWS_SUB, ROWS_SUB)])

    @pl.when(c == 1)
    def _():
        pltpu.sync_copy(deg_sh.at[pl.ds(s * ROWS_SUB, ROWS_SUB)],
                        deg1_out.at[pl.ds(s * ROWS_SUB, ROWS_SUB)])


def _sc_degree(dst2d):
    return pl.kernel(
        _deg_body,
        out_type=[
            jax.ShapeDtypeStruct((NPAD,), _f32),
            jax.ShapeDtypeStruct((NPAD,), _f32),
        ],
        mesh=_sc_mesh(),
        scratch_types=[
            pltpu.VMEM((ERW, 128), _i32),
            pltpu.VMEM((128,), _f32),
            pltpu.VMEM((ROWS_SUB,), _f32),
            pltpu.VMEM_SHARED((NPAD,), _f32),
        ],
    )(dst2d)


_NBUF = 2
_CH = 128
_NCHROWS = EPAD // _CH
_DH = D // 2
_IBLK = 8

_CHUNKS_A = 152
_CHUNKS_B = 8


def _agg_ring(eidx_hbm, xs_hbm, acc_sh, idx_v, rows, gsems, ssems, base,
              nchunks):
    for j in range(nchunks // _IBLK):
        pltpu.sync_copy(
            eidx_hbm.at[pl.ds(2 * (base + j * _IBLK), 2 * _IBLK)], idx_v)
        for i in range(_NBUF):
            pltpu.async_copy(xs_hbm.at[idx_v.at[2 * i]], rows[i],
                             gsems.at[i])

        @pl.loop(0, _IBLK, step=_NBUF)
        def _(t):
            for i in range(_NBUF):
                pltpu.make_async_copy(xs_hbm.at[idx_v.at[2 * (t + i)]],
                                      rows[i], gsems.at[i]).wait()
                pltpu.async_copy(rows[i],
                                 acc_sh.at[idx_v.at[2 * (t + i) + 1]],
                                 ssems.at[i], add=True)
            for i in range(_NBUF):
                pltpu.make_async_copy(rows[i],
                                      acc_sh.at[idx_v.at[2 * (t + i) + 1]],
                                      ssems.at[i]).wait()

                @pl.when(t + _NBUF + i < _IBLK)
                def _():
                    pltpu.async_copy(xs_hbm.at[idx_v.at[2 * (t + _NBUF + i)]],
                                     rows[i], gsems.at[i])


def _agg_body(xs_hbm, eidx_hbm, acc_out, idx_v, rows0, rows1, gsems, ssems,
              acc_sh):
    c = lax.axis_index("c")
    s = lax.axis_index("s")
    rows = [rows0, rows1]

    @pl.loop(0, _CH)
    def _(r):
        @pl.loop(0, 8)
        def _(k):
            rows0[r, pl.ds(k * 16, 16)] = jnp.zeros((16,), _f32)

    for j in range(ROWS_SUB // _CH):
        pltpu.sync_copy(rows0, acc_sh.at[pl.ds(s * ROWS_SUB + j * _CH, _CH)])
    plsc.subcore_barrier()

    @pl.when(c == 0)
    def _():
        _agg_ring(eidx_hbm, xs_hbm, acc_sh, idx_v, rows, gsems, ssems,
                  s * _CHUNKS_A, _CHUNKS_A)

    @pl.when(c == 1)
    def _():
        _agg_ring(eidx_hbm, xs_hbm, acc_sh, idx_v, rows, gsems, ssems,
                  NSUB * _CHUNKS_A + s * _CHUNKS_B, _CHUNKS_B)

    plsc.subcore_barrier()
    pltpu.sync_copy(acc_sh.at[pl.ds(s * ROWS_SUB, ROWS_SUB)],
                    acc_out.at[c, pl.ds(s * ROWS_SUB, ROWS_SUB)])


def _sc_aggregate(xs, eidx2d):
    return pl.kernel(
        _agg_body,
        out_type=jax.ShapeDtypeStruct((NCORE, NPAD, D), _f32),
        mesh=_sc_mesh(),
        scratch_types=[
            pltpu.VMEM((2 * _IBLK, _CH), _i32),
            pltpu.VMEM((_CH, D), _f32),
            pltpu.VMEM((_CH, D), _f32),
            pltpu.SemaphoreType.DMA((_NBUF,)),
            pltpu.SemaphoreType.DMA((_NBUF,)),
            pltpu.VMEM_SHARED((NPAD, D), _f32),
        ],
    )(xs, eidx2d)


def _pool_body(h_hbm, b_hbm, pool_out, cnt0_out, cnt1_out, bidx_v, rows_v,
               ones_v, z_v, pool_sh, cnt_sh):
    c = lax.axis_index("c")
    s = lax.axis_index("s")
    wid = c * NSUB + s

    @pl.loop(0, BCOLS // 16)
    def _(i):
        ones_v[pl.ds(i * 16, 16)] = jnp.ones((16,), _f32)

    @pl.loop(0, CNT_ROWS // 16)
    def _(i):
        z_v[pl.ds(i * 16, 16)] = jnp.zeros((16,), _f32)

    @pl.when(s == 0)
    def _():
        @pl.loop(0, BCOLS)
        def _(r):
            @pl.loop(0, 8)
            def _(k):
                rows_v[r, pl.ds(k * 16, 16)] = jnp.zeros((16,), _f32)

        pltpu.sync_copy(rows_v, pool_sh.at[pl.ds(0, BCOLS)])
        pltpu.sync_copy(rows_v.at[pl.ds(0, POOL_ROWS - BCOLS)],
                        pool_sh.at[pl.ds(BCOLS, POOL_ROWS - BCOLS)])
        pltpu.sync_copy(z_v, cnt_sh)

    plsc.subcore_barrier()

    pltpu.sync_copy(b_hbm.at[wid], bidx_v)

    for k in range(BRW):
        nb = wid * (BRW * BCOLS) + k * BCOLS
        pltpu.sync_copy(h_hbm.at[pl.ds(nb, BCOLS)], rows_v)
        pltpu.sync_copy(rows_v, pool_sh.at[bidx_v.at[k]], add=True)
        pltpu.sync_copy(ones_v, cnt_sh.at[bidx_v.at[k]], add=True)

    plsc.subcore_barrier()

    @pl.when(jnp.logical_and(s == 0, c == 0))
    def _():
        pltpu.sync_copy(pool_sh, pool_out.at[0])
        pltpu.sync_copy(cnt_sh, cnt0_out)

    @pl.when(jnp.logical_and(s == 0, c == 1))
    def _():
        pltpu.sync_copy(pool_sh, pool_out.at[1])
        pltpu.sync_copy(cnt_sh, cnt1_out)


def _sc_pool(h2, batch3d):
    return pl.kernel(
        _pool_body,
        out_type=[
            jax.ShapeDtypeStruct((NCORE, POOL_ROWS, D), _f32),
            jax.ShapeDtypeStruct((CNT_ROWS,), _f32),
            jax.ShapeDtypeStruct((CNT_ROWS,), _f32),
        ],
        mesh=_sc_mesh(),
        scratch_types=[
            pltpu.VMEM((BRW, BCOLS), _i32),
            pltpu.VMEM((BCOLS, D), _f32),
            pltpu.VMEM((BCOLS,), _f32),
            pltpu.VMEM((CNT_ROWS,), _f32),
            pltpu.VMEM_SHARED((POOL_ROWS, D), _f32),
            pltpu.VMEM_SHARED((CNT_ROWS,), _f32),
        ],
    )(h2, batch3d)


_NBLK = 8
_BLK = NPAD // _NBLK


def _dot(a, b):
    return lax.dot_general(a, b, (((1,), (0,)), ((), ())),
                           preferred_element_type=_f32,
                           precision=lax.Precision.HIGHEST)


def _mm_body(x_ref, w_ref, o_ref):
    o_ref[...] = _dot(x_ref[...], w_ref[...])


def _tc_matmul(x, w):
    return pl.pallas_call(
        _mm_body,
        grid=(_NBLK,),
        in_specs=[
            pl.BlockSpec((_BLK, D), lambda i: (i, 0)),
            pl.BlockSpec((D, D), lambda i: (0, 0)),
        ],
        out_specs=pl.BlockSpec((_BLK, D), lambda i: (i, 0)),
        out_shape=jax.ShapeDtypeStruct((NPAD, D), _f32),
    )(x, w)


def _scale_body(xw_ref, deg_ref, xs_ref, dinv_ref):
    deg = deg_ref[0] + deg_ref[1] + 1.0
    dinv = lax.rsqrt(deg)
    dinv_ref[...] = dinv
    xs_ref[...] = xw_ref[...] * dinv


def _tc_scale(xw, deg_pair):
    return pl.pallas_call(
        _scale_body,
        grid=(_NBLK,),
        in_specs=[
            pl.BlockSpec((_BLK, D), lambda i: (i, 0)),
            pl.BlockSpec((NCORE, _BLK, 1), lambda i: (0, i, 0)),
        ],
        out_specs=[
            pl.BlockSpec((_BLK, D), lambda i: (i, 0)),
            pl.BlockSpec((_BLK, 1), lambda i: (i, 0)),
        ],
        out_shape=[
            jax.ShapeDtypeStruct((NPAD, D), _f32),
            jax.ShapeDtypeStruct((NPAD, 1), _f32),
        ],
    )(xw, deg_pair)


def _mid_body(acc_ref, xs_ref, dinv_ref, b_ref, w_ref, xs2_ref):
    dinv = dinv_ref[...]
    h = (acc_ref[0] + acc_ref[1] + xs_ref[...]) * dinv + b_ref[...]
    h = jnp.maximum(h, 0.0)
    xs2_ref[...] = _dot(h, w_ref[...]) * dinv


def _tc_mid(acc, xs, dinv, b, w):
    return pl.pallas_call(
        _mid_body,
        grid=(_NBLK,),
        in_specs=[
            pl.BlockSpec((NCORE, _BLK, D), lambda i: (0, i, 0)),
            pl.BlockSpec((_BLK, D), lambda i: (i, 0)),
            pl.BlockSpec((_BLK, 1), lambda i: (i, 0)),
            pl.BlockSpec((1, D), lambda i: (0, 0)),
            pl.BlockSpec((D, D), lambda i: (0, 0)),
        ],
        out_specs=pl.BlockSpec((_BLK, D), lambda i: (i, 0)),
        out_shape=jax.ShapeDtypeStruct((NPAD, D), _f32),
    )(acc, xs, dinv, b, w)


def _final_node_body(acc_ref, xs_ref, dinv_ref, b_ref, h_ref):
    h = (acc_ref[0] + acc_ref[1] + xs_ref[...]) * dinv_ref[...] + b_ref[...]
    h_ref[...] = jnp.maximum(h, 0.0)


def _tc_final_nodes(acc, xs, dinv, b):
    return pl.pallas_call(
        _final_node_body,
        grid=(_NBLK,),
        in_specs=[
            pl.BlockSpec((NCORE, _BLK, D), lambda i: (0, i, 0)),
            pl.BlockSpec((_BLK, D), lambda i: (i, 0)),
            pl.BlockSpec((_BLK, 1), lambda i: (i, 0)),
            pl.BlockSpec((1, D), lambda i: (0, 0)),
        ],
        out_specs=pl.BlockSpec((_BLK, D), lambda i: (i, 0)),
        out_shape=jax.ShapeDtypeStruct((NPAD, D), _f32),
    )(acc, xs, dinv, b)


def _head_body(pool_ref, cnt_ref, wl_ref, bl_ref, y_ref):
    ps = pool_ref[0, :G, :] + pool_ref[1, :G, :]
    cn = cnt_ref[0, :G, :] + cnt_ref[1, :G, :]
    cn = jnp.maximum(cn, 1.0)
    pm = ps / cn
    y_ref[...] = _dot(pm, wl_ref[...]) + bl_ref[...]


def _tc_head(pool, cnt, wl, bl):
    return pl.pallas_call(
        _head_body,
        out_shape=jax.ShapeDtypeStruct((G, 2), _f32),
    )(pool, cnt, wl, bl)


def kernel(x, edge_index, batch, W1, b1, W2, b2, Wl, bl):
    src = edge_index[0].astype(_i32)
    dst = edge_index[1].astype(_i32)
    src_p = jnp.concatenate([src, jnp.zeros((EPAD - E,), _i32)])
    dst_p = jnp.concatenate([dst, jnp.full((EPAD - E,), DUMMY_DST, _i32)])
    dst2d = dst_p.reshape(EROWS, 128)
    eidx2d = jnp.stack(
        [src_p.reshape(_NCHROWS, _CH), dst_p.reshape(_NCHROWS, _CH)],
        axis=1).reshape(2 * _NCHROWS, _CH)
    batch3d = jnp.concatenate(
        [batch.astype(_i32), jnp.full((NPAD - N,), G, _i32)]
    ).reshape(NW, BRW, BCOLS)
    x_pad = jnp.pad(x, ((0, NPAD - N), (0, 0)))

    deg0, deg1 = _sc_degree(dst2d)
    xw1 = _tc_matmul(x_pad, W1)
    deg3 = jnp.stack([deg0, deg1]).reshape(NCORE, NPAD, 1)
    xs1, dinv = _tc_scale(xw1, deg3)

    acc1 = _sc_aggregate(xs1, eidx2d)
    xs2 = _tc_mid(acc1, xs1, dinv, b1.reshape(1, D), W2)

    acc2 = _sc_aggregate(xs2, eidx2d)
    h2 = _tc_final_nodes(acc2, xs2, dinv, b2.reshape(1, D))

    pool, cnt0, cnt1 = _sc_pool(h2, batch3d)
    cnt = jnp.stack([cnt0, cnt1]).reshape(NCORE, CNT_ROWS, 1)
    y = _tc_head(pool, cnt, Wl, bl.reshape(1, 2))
    return y
